# baseline (speedup 1.0000x reference)
import functools

import jax
import jax.numpy as jnp
from jax import lax
from jax.experimental import pallas as pl
from jax.experimental.pallas import tpu as pltpu
from jax.experimental.pallas import tpu_sc as plsc

N_NODES = 10000
D = 128

NC = 2
NS = 16

HS = 5120
H_PAD = NC * HS
H_STAGE = HS // NS

CHUNK = 32
CHUNKS_PER_TILE = 640
EPT = CHUNK * CHUNKS_PER_TILE
E_PAD = EPT * NS

ROWS_PER_TILE = 632
N_PAD = ROWS_PER_TILE * NS

BM = 1000


def _mm_body(x_ref, deg_ref, w_ref, o_ref):
    o_ref[...] = jnp.dot(x_ref[...] * deg_ref[...], w_ref[...],
                         preferred_element_type=jnp.float32)


def _matmul(x, deg, w):
    grid = N_NODES // BM
    return pl.pallas_call(
        _mm_body,
        grid=(grid,),
        in_specs=[
            pl.BlockSpec((BM, D), lambda i: (i, 0)),
            pl.BlockSpec((BM, 1), lambda i: (i, 0)),
            pl.BlockSpec((D, D), lambda i: (0, 0)),
        ],
        out_specs=pl.BlockSpec((BM, D), lambda i: (i, 0)),
        out_shape=jax.ShapeDtypeStruct((N_NODES, D), jnp.float32),
    )(x, deg, w)


def _sc_aggregate(h2, idx3, zeros):
    mesh = plsc.VectorSubcoreMesh(core_axis_name="c", subcore_axis_name="s")
    last = CHUNKS_PER_TILE // 2 - 1

    @functools.partial(
        pl.kernel,
        out_type=jax.ShapeDtypeStruct((NC, N_PAD, D), jnp.float32),
        mesh=mesh,
        scratch_types=[
            pltpu.VMEM_SHARED((HS, D), jnp.float32),
            pltpu.VMEM_SHARED((N_PAD, D), jnp.float32),
            pltpu.VMEM((2, CHUNK), jnp.int32),
            pltpu.VMEM((2, CHUNK), jnp.int32),
            pltpu.VMEM((CHUNK, D), jnp.float32),
            pltpu.VMEM((CHUNK, D), jnp.float32),
            pltpu.SemaphoreType.DMA,
            pltpu.SemaphoreType.DMA,
            pltpu.SemaphoreType.DMA,
            pltpu.SemaphoreType.DMA,
        ],
    )
    def k(h_hbm, idx_hbm, zeros_hbm, out_hbm,
          hsp, acc, isl0, isl1, rows_a, rows_b, sem_i0, sem_i1, sem_a, sem_b):
        cid = lax.axis_index("c")
        sid = lax.axis_index("s")

        pltpu.sync_copy(h_hbm.at[pl.ds(cid * HS + sid * H_STAGE, H_STAGE)],
                        hsp.at[pl.ds(sid * H_STAGE, H_STAGE)])
        pltpu.sync_copy(zeros_hbm, acc.at[pl.ds(sid * ROWS_PER_TILE, ROWS_PER_TILE)])
        plsc.subcore_barrier()

        pltpu.sync_copy(idx_hbm.at[cid, sid, 0], isl0)
        pltpu.async_copy(hsp.at[isl0.at[0]], rows_a, sem_a)
        pltpu.async_copy(idx_hbm.at[cid, sid, 1], isl1, sem_i1)

        def body(j, _):
            g = 2 * j
            pltpu.make_async_copy(idx_hbm.at[cid, sid, g + 1], isl1, sem_i1).wait()
            pltpu.async_copy(hsp.at[isl1.at[0]], rows_b, sem_b)

            pltpu.make_async_copy(hsp.at[isl0.at[0]], rows_a, sem_a).wait()
            pltpu.sync_copy(rows_a, acc.at[isl0.at[1]], add=True)

            @pl.when(j != last)
            def _next_even():
                pltpu.sync_copy(idx_hbm.at[cid, sid, g + 2], isl0)
                pltpu.async_copy(hsp.at[isl0.at[0]], rows_a, sem_a)

            pltpu.make_async_copy(hsp.at[isl1.at[0]], rows_b, sem_b).wait()
            pltpu.sync_copy(rows_b, acc.at[isl1.at[1]], add=True)

            @pl.when(j != last)
            def _next_odd():
                pltpu.async_copy(idx_hbm.at[cid, sid, g + 3], isl1, sem_i1)

            return _

        lax.fori_loop(0, CHUNKS_PER_TILE // 2, body, None)

        plsc.subcore_barrier()
        pltpu.sync_copy(acc.at[pl.ds(sid * ROWS_PER_TILE, ROWS_PER_TILE)],
                        out_hbm.at[cid, pl.ds(sid * ROWS_PER_TILE, ROWS_PER_TILE)])

    return k(h2, idx3, zeros)


def _comb_body(p_ref, deg_ref, b_ref, o_ref):
    o_ref[...] = (p_ref[0] + p_ref[1]) * deg_ref[...] + b_ref[...]


def _combine(partials, deg, bias):
    grid = N_NODES // BM
    return pl.pallas_call(
        _comb_body,
        grid=(grid,),
        in_specs=[
            pl.BlockSpec((NC, BM, D), lambda i: (0, i, 0)),
            pl.BlockSpec((BM, 1), lambda i: (i, 0)),
            pl.BlockSpec((1, D), lambda i: (0, 0)),
        ],
        out_specs=pl.BlockSpec((BM, D), lambda i: (i, 0)),
        out_shape=jax.ShapeDtypeStruct((N_NODES, D), jnp.float32),
    )(partials, deg, bias)


def kernel(x, edge_index, deg_inv_sqrt, weight, bias):
    src = edge_index[0].astype(jnp.int32)
    dst = edge_index[1].astype(jnp.int32)
    n_extra = E_PAD - src.shape[0]
    src = jnp.concatenate([src, jnp.zeros((n_extra,), jnp.int32)])
    dst = jnp.concatenate([dst, jnp.full((n_extra,), N_NODES, jnp.int32)])

    on0 = src < HS
    src0 = jnp.where(on0, src, 0)
    dst0 = jnp.where(on0, dst, N_NODES)
    src1 = jnp.where(on0, 0, src - HS)
    dst1 = jnp.where(on0, N_NODES, dst)
    idx3 = jnp.stack(
        [jnp.stack([src0.reshape(NS, CHUNKS_PER_TILE, CHUNK),
                    dst0.reshape(NS, CHUNKS_PER_TILE, CHUNK)], axis=2),
         jnp.stack([src1.reshape(NS, CHUNKS_PER_TILE, CHUNK),
                    dst1.reshape(NS, CHUNKS_PER_TILE, CHUNK)], axis=2)], axis=0)

    deg2d = deg_inv_sqrt[:, None]
    h = _matmul(x, deg2d, weight)
    h2 = jnp.pad(h, ((0, H_PAD - N_NODES), (0, 0)))
    zeros = jnp.zeros((ROWS_PER_TILE, D), jnp.float32)
    partials = _sc_aggregate(h2, idx3, zeros)
    return _combine(partials, deg2d, bias.reshape(1, D))


# speedup vs baseline: 1.3000x; 1.3000x over previous
import functools

import jax
import jax.numpy as jnp
from jax import lax
from jax.experimental import pallas as pl
from jax.experimental.pallas import tpu as pltpu
from jax.experimental.pallas import tpu_sc as plsc

N_NODES = 10000
D = 128

NC = 2
NS = 16
NW = NC * NS

CHUNK = 64
CHUNKS_PER_TILE = 160
EPT = CHUNK * CHUNKS_PER_TILE
E_PAD = EPT * NW

NBUF = 4
NIDX = 8
STEPS = CHUNKS_PER_TILE + NBUF
GROUPS = (STEPS + NIDX - 1) // NIDX

ROWS_PER_TILE = 632
N_PAD = ROWS_PER_TILE * NS

BM = 1000


def _mm_body(x_ref, deg_ref, w_ref, o_ref):
    o_ref[...] = jnp.dot(x_ref[...] * deg_ref[...], w_ref[...],
                         preferred_element_type=jnp.float32)


def _matmul(x, deg, w):
    grid = N_NODES // BM
    return pl.pallas_call(
        _mm_body,
        grid=(grid,),
        in_specs=[
            pl.BlockSpec((BM, D), lambda i: (i, 0)),
            pl.BlockSpec((BM, 1), lambda i: (i, 0)),
            pl.BlockSpec((D, D), lambda i: (0, 0)),
        ],
        out_specs=pl.BlockSpec((BM, D), lambda i: (i, 0)),
        out_shape=jax.ShapeDtypeStruct((N_NODES, D), jnp.float32),
    )(x, deg, w)


def _sc_aggregate(h, idx2, zeros):
    mesh = plsc.VectorSubcoreMesh(core_axis_name="c", subcore_axis_name="s")

    @functools.partial(
        pl.kernel,
        out_type=jax.ShapeDtypeStruct((NC, N_PAD, D), jnp.float32),
        mesh=mesh,
        scratch_types=[
            pltpu.VMEM_SHARED((N_PAD, D), jnp.float32),
            pltpu.VMEM((NIDX, 2, CHUNK), jnp.int32),
            pltpu.VMEM((NBUF, CHUNK, D), jnp.float32),
        ] + [pltpu.SemaphoreType.DMA] * (NBUF + NIDX),
    )
    def k(h_hbm, idx_hbm, zeros_hbm, out_hbm, acc, isl, rows, *sems):
        sem_g = sems[:NBUF]
        sem_i = sems[NBUF:]
        cid = lax.axis_index("c")
        sid = lax.axis_index("s")
        wid = cid * NS + sid

        pltpu.sync_copy(zeros_hbm, acc.at[pl.ds(sid * ROWS_PER_TILE, ROWS_PER_TILE)])
        plsc.subcore_barrier()

        for c in range(NBUF):
            pltpu.async_copy(idx_hbm.at[wid, c], isl.at[c], sem_i[c])

        def body(g, _):
            for u in range(NIDX):
                s = g * NIDX + u
                b = u % NBUF
                jp = (u - NBUF) % NIDX

                @pl.when(jnp.logical_and(s >= NBUF, s < CHUNKS_PER_TILE + NBUF))
                def _retire():
                    pltpu.make_async_copy(h_hbm.at[isl.at[jp, 0]], rows.at[b],
                                          sem_g[b]).wait()
                    pltpu.sync_copy(rows.at[b], acc.at[isl.at[jp, 1]], add=True)

                @pl.when(s + NBUF < CHUNKS_PER_TILE)
                def _prefetch():
                    pltpu.async_copy(idx_hbm.at[wid, s + NBUF], isl.at[jp],
                                     sem_i[jp])

                @pl.when(s < CHUNKS_PER_TILE)
                def _launch():
                    pltpu.make_async_copy(idx_hbm.at[wid, 0], isl.at[u],
                                          sem_i[u]).wait()
                    pltpu.async_copy(h_hbm.at[isl.at[u, 0]], rows.at[b], sem_g[b])

            return _

        lax.fori_loop(0, GROUPS, body, None)

        plsc.subcore_barrier()
        pltpu.sync_copy(acc.at[pl.ds(sid * ROWS_PER_TILE, ROWS_PER_TILE)],
                        out_hbm.at[cid, pl.ds(sid * ROWS_PER_TILE, ROWS_PER_TILE)])

    return k(h, idx2, zeros)


def _comb_body(p_ref, deg_ref, b_ref, o_ref):
    o_ref[...] = (p_ref[0] + p_ref[1]) * deg_ref[...] + b_ref[...]


def _combine(partials, deg, bias):
    grid = N_NODES // BM
    return pl.pallas_call(
        _comb_body,
        grid=(grid,),
        in_specs=[
            pl.BlockSpec((NC, BM, D), lambda i: (0, i, 0)),
            pl.BlockSpec((BM, 1), lambda i: (i, 0)),
            pl.BlockSpec((1, D), lambda i: (0, 0)),
        ],
        out_specs=pl.BlockSpec((BM, D), lambda i: (i, 0)),
        out_shape=jax.ShapeDtypeStruct((N_NODES, D), jnp.float32),
    )(partials, deg, bias)


def kernel(x, edge_index, deg_inv_sqrt, weight, bias):
    src = edge_index[0].astype(jnp.int32)
    dst = edge_index[1].astype(jnp.int32)
    n_extra = E_PAD - src.shape[0]
    src = jnp.concatenate([src, jnp.zeros((n_extra,), jnp.int32)])
    dst = jnp.concatenate([dst, jnp.full((n_extra,), N_NODES, jnp.int32)])
    idx2 = jnp.stack(
        [src.reshape(NW, CHUNKS_PER_TILE, CHUNK),
         dst.reshape(NW, CHUNKS_PER_TILE, CHUNK)], axis=2)

    deg2d = deg_inv_sqrt[:, None]
    h = _matmul(x, deg2d, weight)
    zeros = jnp.zeros((ROWS_PER_TILE, D), jnp.float32)
    partials = _sc_aggregate(h, idx2, zeros)
    return _combine(partials, deg2d, bias.reshape(1, D))
